# baseline (speedup 1.0000x reference)
import functools
import math

import jax
import jax.numpy as jnp
from jax import lax
from jax.experimental import pallas as pl
from jax.experimental.pallas import tpu as pltpu
from jax.experimental.pallas import tpu_sc as plsc

_TOKEN_EMBED_DIM = 64
_SCALE = math.sqrt(_TOKEN_EMBED_DIM)

_NC = 2
_NS = 16
_NW = _NC * _NS

_CHUNK = 512
_IDXW = 128
_K = _CHUNK // _IDXW


def _make_embed(n_rows, vocab, d):
    assert n_rows % (_NW * _CHUNK) == 0
    b_per_w = n_rows // _NW
    nchunk = b_per_w // _CHUNK
    mesh = plsc.VectorSubcoreMesh(core_axis_name="c", subcore_axis_name="s")

    @functools.partial(
        pl.kernel,
        mesh=mesh,
        compiler_params=pltpu.CompilerParams(use_tc_tiling_on_sc=False),
        out_type=jax.ShapeDtypeStruct((n_rows, d), jnp.float32),
        scratch_types=[
            pltpu.VMEM((_K, _IDXW), jnp.int32),
            pltpu.VMEM((_CHUNK, d), jnp.float32),
            pltpu.SemaphoreType.DMA,
        ],
    )
    def embed(idx_hbm, table_hbm, out_hbm, idx_v, rows_v, gsem):
        wid = lax.axis_index("s") * _NC + lax.axis_index("c")
        idx_row0 = wid * (nchunk * _K)
        out_row0 = wid * b_per_w

        def chunk_body(g, carry):
            pltpu.sync_copy(idx_hbm.at[pl.ds(idx_row0 + g * _K, _K)], idx_v)
            descs = [
                pltpu.async_copy(
                    table_hbm.at[idx_v.at[j]],
                    rows_v.at[pl.ds(j * _IDXW, _IDXW)],
                    gsem,
                )
                for j in range(_K)
            ]
            for dsc in descs:
                dsc.wait()

            def scale_body(r, c2):
                for u in range(4):
                    for s in range(d // 16):
                        sl = pl.ds(s * 16, 16)
                        rows_v[r * 4 + u, sl] = rows_v[r * 4 + u, sl] * _SCALE
                return c2

            lax.fori_loop(0, _CHUNK // 4, scale_body, 0, unroll=False)
            pltpu.sync_copy(
                rows_v, out_hbm.at[pl.ds(out_row0 + g * _CHUNK, _CHUNK)]
            )
            return carry

        lax.fori_loop(0, nchunk, chunk_body, 0, unroll=False)

    return embed


def kernel(x, table):
    lead_shape = x.shape
    n_rows = x.size
    vocab, d = table.shape
    idx = x.reshape(-1, _IDXW).astype(jnp.int32)
    out = _make_embed(n_rows, vocab, d)(idx, table)
    return out.reshape(*lead_shape, d)


# speedup vs baseline: 1.0959x; 1.0959x over previous
import functools
import math

import jax
import jax.numpy as jnp
from jax import lax
from jax.experimental import pallas as pl
from jax.experimental.pallas import tpu as pltpu
from jax.experimental.pallas import tpu_sc as plsc

_TOKEN_EMBED_DIM = 64
_SCALE = math.sqrt(_TOKEN_EMBED_DIM)

_NC = 2
_NS = 16
_NW = _NC * _NS

_CHUNK = 512
_IDXW = 128
_K = _CHUNK // _IDXW
_NBUF = 2


def _make_embed(n_rows, vocab, d):
    assert n_rows % (_NW * _CHUNK * _NBUF) == 0
    b_per_w = n_rows // _NW
    nchunk = b_per_w // _CHUNK
    nouter = nchunk // _NBUF
    mesh = plsc.VectorSubcoreMesh(core_axis_name="c", subcore_axis_name="s")

    @functools.partial(
        pl.kernel,
        mesh=mesh,
        compiler_params=pltpu.CompilerParams(use_tc_tiling_on_sc=False),
        out_type=jax.ShapeDtypeStruct((n_rows, d), jnp.float32),
        scratch_types=[
            pltpu.VMEM((_NBUF, _K, _IDXW), jnp.int32),
            pltpu.VMEM((_NBUF * _CHUNK, d), jnp.float32),
            pltpu.SemaphoreType.DMA,
            pltpu.SemaphoreType.DMA,
            pltpu.SemaphoreType.DMA,
            pltpu.SemaphoreType.DMA,
            pltpu.SemaphoreType.DMA,
            pltpu.SemaphoreType.DMA,
        ],
    )
    def embed(idx_hbm, table_hbm, out_hbm, idx_v, rows_v, g0, g1, o0, o1,
              i0, i1):
        gsems, osems, isems = (g0, g1), (o0, o1), (i0, i1)
        wid = lax.axis_index("s") * _NC + lax.axis_index("c")
        idx_row0 = wid * (nchunk * _K)
        out_row0 = wid * b_per_w

        def rows_sl(b):
            return rows_v.at[pl.ds(b * _CHUNK, _CHUNK)]

        def fire_gathers(b):
            for j in range(_K):
                pltpu.async_copy(
                    table_hbm.at[idx_v.at[b, j]],
                    rows_v.at[pl.ds(b * _CHUNK + j * _IDXW, _IDXW)],
                    gsems[b],
                )

        def drain_gathers(b):
            pltpu.make_async_copy(
                table_hbm.at[pl.ds(0, _CHUNK)], rows_sl(b), gsems[b]
            ).wait()

        def fire_idx(b, n):
            pltpu.async_copy(
                idx_hbm.at[pl.ds(idx_row0 + n * _K, _K)], idx_v.at[b],
                isems[b],
            )

        def wait_idx(b):
            pltpu.make_async_copy(
                idx_hbm.at[pl.ds(0, _K)], idx_v.at[b], isems[b]
            ).wait()

        def fire_out(b, n):
            pltpu.async_copy(
                rows_sl(b), out_hbm.at[pl.ds(out_row0 + n * _CHUNK, _CHUNK)],
                osems[b],
            )

        def wait_out(b):
            pltpu.make_async_copy(
                rows_sl(b), out_hbm.at[pl.ds(0, _CHUNK)], osems[b]
            ).wait()

        def scale(b):
            base = b * _CHUNK

            def body(r, c):
                for u in range(4):
                    row = base + r * 4 + u
                    for s in range(d // 16):
                        sl = pl.ds(s * 16, 16)
                        rows_v[row, sl] = rows_v[row, sl] * _SCALE
                return c

            lax.fori_loop(0, _CHUNK // 4, body, 0, unroll=False)

        def visit(n, b, prefetch=True, fire_next=True, first=False):
            drain_gathers(b)
            if prefetch:
                fire_idx(b, n + _NBUF)
            scale(b)
            fire_out(b, n)
            if fire_next:
                bp = (b - 1) % _NBUF
                if not first:
                    wait_out(bp)
                wait_idx(bp)
                fire_gathers(bp)

        pltpu.sync_copy(idx_hbm.at[pl.ds(idx_row0, _K)], idx_v.at[0])
        fire_gathers(0)
        fire_idx(1, 1)

        visit(0, 0, first=True)
        for b in range(1, _NBUF):
            visit(b, b)

        def group(g2, c):
            n0 = g2 * _NBUF
            for b in range(_NBUF):
                visit(n0 + b, b)
            return c

        lax.fori_loop(1, nouter - 1, group, 0, unroll=False)

        n0 = (nouter - 1) * _NBUF
        visit(n0, 0, prefetch=False)
        for b in range(1, _NBUF - 1):
            visit(n0 + b, b, prefetch=False)
        bl = _NBUF - 1
        drain_gathers(bl)
        scale(bl)
        fire_out(bl, n0 + bl)
        for b in range(_NBUF):
            wait_out(b)

    return embed


def kernel(x, table):
    lead_shape = x.shape
    n_rows = x.size
    vocab, d = table.shape
    idx = x.reshape(-1, _IDXW).astype(jnp.int32)
    out = _make_embed(n_rows, vocab, d)(idx, table)
    return out.reshape(*lead_shape, d)
